# baseline (speedup 1.0000x reference)
import functools

import jax
import jax.numpy as jnp
from jax import lax
from jax.experimental import pallas as pl
from jax.experimental.pallas import tpu as pltpu
from jax.experimental.pallas import tpu_sc as plsc

N = 10000
NPAD = 10240
D = 128
E = 320000
NC = 2
NS = 16
NW = NC * NS
VPT = NPAD // NW
ACCR = VPT + 8
CHUNK = 128
BINCAP = 11264
BCH = BINCAP // CHUNK
BCHUNK = 4000
EBCH = E // BCHUNK

_mesh = functools.partial(
    plsc.VectorSubcoreMesh, core_axis_name="c", subcore_axis_name="s",
    num_cores=NC, num_subcores=NS)


@functools.partial(
    pl.kernel,
    out_type=(
        jax.ShapeDtypeStruct((NW, BINCAP), jnp.int32),
        jax.ShapeDtypeStruct((NW, BINCAP), jnp.int32),
        jax.ShapeDtypeStruct((NW, VPT), jnp.float32),
    ),
    mesh=_mesh(),
    compiler_params=pltpu.CompilerParams(needs_layout_passes=False),
    scratch_types=[
        pltpu.VMEM((BCHUNK,), jnp.int32),
        pltpu.VMEM((BCHUNK,), jnp.int32),
        pltpu.VMEM((BCHUNK,), jnp.int32),
        pltpu.VMEM((BCHUNK,), jnp.int32),
        pltpu.VMEM((BINCAP,), jnp.int32),
        pltpu.VMEM((BINCAP,), jnp.int32),
        pltpu.VMEM((VPT,), jnp.float32),
        pltpu.SemaphoreType.DMA,
        pltpu.SemaphoreType.DMA,
    ],
)
def _bin_kernel(src_hbm, dst_hbm, bsrc_out, bdst_out, deg_out,
                sbuf0, sbuf1, dbuf0, dbuf1, bsrc_l, bdst_l, deg_l,
                sem0, sem1):
    c = lax.axis_index("c")
    s = lax.axis_index("s")
    w = c * NS + s
    base = w * VPT
    sems = (sem0, sem1)
    sbufs = (sbuf0, sbuf1)
    dbufs = (dbuf0, dbuf1)

    zi = jnp.zeros((16,), jnp.int32)
    ti = jnp.full((16,), VPT, jnp.int32)
    zf = jnp.zeros((16,), jnp.float32)
    onef = jnp.ones((16,), jnp.float32)

    def _fire(k, p):
        pltpu.async_copy(
            src_hbm.at[pl.ds(k * BCHUNK, BCHUNK)], sbufs[p], sems[p])
        pltpu.async_copy(
            dst_hbm.at[pl.ds(k * BCHUNK, BCHUNK)], dbufs[p], sems[p])

    def _drain(p):
        pltpu.make_async_copy(
            src_hbm.at[pl.ds(0, BCHUNK)], sbufs[p], sems[p]).wait()
        pltpu.make_async_copy(
            dst_hbm.at[pl.ds(0, BCHUNK)], dbufs[p], sems[p]).wait()

    _fire(0, 0)
    _fire(1, 1)

    def pre(i, carry):
        bsrc_l[pl.ds(i * 16, 16)] = zi
        bdst_l[pl.ds(i * 16, 16)] = ti
        return carry

    lax.fori_loop(0, BINCAP // 16, pre, 0)

    def zdeg(i, carry):
        deg_l[pl.ds(i * 16, 16)] = zf
        return carry

    lax.fori_loop(0, VPT // 16, zdeg, 0)

    def _consume(p, cnt):
        for j in range(BCHUNK // 16):
            s16 = sbufs[p][pl.ds(j * 16, 16)]
            d16 = dbufs[p][pl.ds(j * 16, 16)]
            dl16 = d16 - base
            m = (dl16 >= 0) & (dl16 < VPT)
            dlc = jnp.where(m, dl16, 0)
            plsc.addupdate_scatter(deg_l, [dlc], onef, mask=m)
            plsc.store_compressed(bsrc_l.at[pl.ds(cnt, 16)], s16, mask=m)
            plsc.store_compressed(
                bdst_l.at[pl.ds(cnt, 16)], jnp.where(m, dl16, ti), mask=m)
            cnt = jnp.minimum(cnt + jnp.sum(m.astype(jnp.int32)), BINCAP - 16)
        return cnt

    def block(i, cnt):
        for p in range(2):
            k = i * 2 + p
            _drain(p)
            cnt = _consume(p, cnt)

            @pl.when(k + 2 < EBCH)
            def _():
                _fire(k + 2, p)
        return cnt

    lax.fori_loop(0, EBCH // 2, block, jnp.int32(0))

    pltpu.sync_copy(bsrc_l, bsrc_out.at[w])
    pltpu.sync_copy(bdst_l, bdst_out.at[w])
    pltpu.sync_copy(deg_l, deg_out.at[w])


@functools.partial(
    pl.kernel,
    out_type=jax.ShapeDtypeStruct((NPAD, D), jnp.float32),
    mesh=_mesh(),
    compiler_params=pltpu.CompilerParams(needs_layout_passes=False),
    scratch_types=[
        pltpu.VMEM((BINCAP,), jnp.int32),
        pltpu.VMEM((BINCAP,), jnp.int32),
        pltpu.VMEM((CHUNK, D), jnp.float32),
        pltpu.VMEM((CHUNK, D), jnp.float32),
        pltpu.VMEM((ACCR, D), jnp.float32),
        pltpu.SemaphoreType.DMA,
        pltpu.SemaphoreType.DMA,
        pltpu.SemaphoreType.DMA,
    ],
)
def _spmm_kernel(bsrc_hbm, bdst_hbm, g_hbm, out_hbm,
                 sball, dball, rows0, rows1, acc, semi, sem0, sem1):
    c = lax.axis_index("c")
    s = lax.axis_index("s")
    w = c * NS + s
    sems = (sem0, sem1)
    rowss = (rows0, rows1)

    pltpu.async_copy(bsrc_hbm.at[w], sball, semi)
    pltpu.async_copy(bdst_hbm.at[w], dball, semi)

    zf = jnp.zeros((16,), jnp.float32)

    def zacc(i, carry):
        r = i // (D // 16)
        col = (i % (D // 16)) * 16
        acc[r, pl.ds(col, 16)] = zf
        return carry

    lax.fori_loop(0, ACCR * (D // 16), zacc, 0)

    pltpu.make_async_copy(bsrc_hbm.at[w], sball, semi).wait()
    pltpu.make_async_copy(bdst_hbm.at[w], dball, semi).wait()

    QR = CHUNK // 4

    def _fire(k, p):
        for q in range(4):
            pltpu.async_copy(
                g_hbm.at[sball.at[pl.ds(k * CHUNK + q * QR, QR)]],
                rowss[p].at[pl.ds(q * QR, QR)], sems[p])

    def _drain(p):
        for q in range(4):
            pltpu.make_async_copy(
                g_hbm.at[sball.at[pl.ds(0, QR)]],
                rowss[p].at[pl.ds(q * QR, QR)], sems[p]).wait()

    _fire(0, 0)
    _fire(1, 1)

    iota16 = lax.iota(jnp.int32, 16)

    def _consume(k, p):
        rws = rowss[p]

        def jbody(j, carry):
            dl16 = dball[pl.ds(k * CHUNK + j * 16, 16)]
            e16 = j * 16 + iota16
            for ec in range(D):
                ecv = jnp.full((16,), ec, jnp.int32)
                vals = plsc.load_gather(rws, [e16, ecv])
                plsc.addupdate_scatter(acc, [dl16, ecv], vals)
            return carry

        lax.fori_loop(0, CHUNK // 16, jbody, 0)

    def chunk2(i, carry):
        for p in range(2):
            k = i * 2 + p
            _drain(p)
            _consume(k, p)

            @pl.when(k + 2 < BCH)
            def _():
                _fire(k + 2, p)
        return carry

    lax.fori_loop(0, BCH // 2, chunk2, 0)

    pltpu.sync_copy(acc.at[pl.ds(0, VPT)], out_hbm.at[pl.ds(w * VPT, VPT)])


_RBLK = 1000
_GRID = N // _RBLK

_row_spec = pl.BlockSpec((_RBLK, D), lambda i: (i, 0))
_col_spec = pl.BlockSpec((_RBLK, 1), lambda i: (i, 0))
_w_spec = pl.BlockSpec((D, D), lambda i: (0, 0))
_b_spec = pl.BlockSpec((1, D), lambda i: (0, 0))


def _k1_body(deg_ref, x_ref, w_ref, dinv_ref, g_ref):
    deg = deg_ref[...] + 1.0
    dinv = lax.rsqrt(jnp.maximum(deg, 1.0))
    dinv_ref[...] = dinv
    g_ref[...] = dinv * jnp.dot(x_ref[...], w_ref[...],
                                preferred_element_type=jnp.float32)


_k1 = pl.pallas_call(
    _k1_body,
    grid=(_GRID,),
    in_specs=[_col_spec, _row_spec, _w_spec],
    out_specs=[_col_spec, _row_spec],
    out_shape=[jax.ShapeDtypeStruct((N, 1), jnp.float32),
               jax.ShapeDtypeStruct((N, D), jnp.float32)],
)


def _kmid_body(p_ref, g_ref, dinv_ref, b_ref, w_ref, gn_ref):
    dinv = dinv_ref[...]
    h = dinv * (p_ref[...] + g_ref[...]) + b_ref[...]
    h = jnp.maximum(h, 0.0)
    gn_ref[...] = dinv * jnp.dot(h, w_ref[...],
                                 preferred_element_type=jnp.float32)


_kmid = pl.pallas_call(
    _kmid_body,
    grid=(_GRID,),
    in_specs=[_row_spec, _row_spec, _col_spec, _b_spec, _w_spec],
    out_specs=_row_spec,
    out_shape=jax.ShapeDtypeStruct((N, D), jnp.float32),
)


def _kfin_body(p_ref, g_ref, dinv_ref, b_ref, wo_ref, bo_ref,
               h_ref, out_ref):
    h = dinv_ref[...] * (p_ref[...] + g_ref[...]) + b_ref[...]
    h_ref[...] = h
    out_ref[...] = jnp.dot(h, wo_ref[...],
                           preferred_element_type=jnp.float32) + bo_ref[...]


_kfin = pl.pallas_call(
    _kfin_body,
    grid=(_GRID,),
    in_specs=[_row_spec, _row_spec, _col_spec, _b_spec, _w_spec, _b_spec],
    out_specs=[_row_spec, _row_spec],
    out_shape=[jax.ShapeDtypeStruct((N, D), jnp.float32),
               jax.ShapeDtypeStruct((N, D), jnp.float32)],
)


def kernel(x, edge_index, W1, b1, Wh1, bh1, W2, b2, Wout, bout):
    src = edge_index[0]
    dst = edge_index[1]

    bsrc, bdst, degp = _bin_kernel(src, dst)
    deg = degp.reshape(NPAD, 1)

    dinv, g1 = _k1(deg, x, W1)
    p1 = _spmm_kernel(bsrc, bdst, g1)
    g2 = _kmid(p1, g1, dinv, b1.reshape(1, D), Wh1)
    p2 = _spmm_kernel(bsrc, bdst, g2)
    g3 = _kmid(p2, g2, dinv, bh1.reshape(1, D), W2)
    p3 = _spmm_kernel(bsrc, bdst, g3)

    wout_pad = jnp.zeros((D, D), jnp.float32).at[:, :Wout.shape[1]].set(Wout)
    bout_pad = jnp.zeros((1, D), jnp.float32).at[0, :bout.shape[0]].set(bout)
    h3, out_pad = _kfin(p3, g3, dinv, b2.reshape(1, D), wout_pad, bout_pad)
    return (out_pad[:, :Wout.shape[1]], h3)


# speedup vs baseline: 1.7714x; 1.7714x over previous
import functools

import jax
import jax.numpy as jnp
from jax import lax
from jax.experimental import pallas as pl
from jax.experimental.pallas import tpu as pltpu
from jax.experimental.pallas import tpu_sc as plsc

N = 10000
NPAD = 10240
D = 128
E = 320000
NC = 2
NS = 16
NW = NC * NS
VPT = NPAD // NW
ACCR = VPT + 8
CHUNK = 128
BINCAP = 11264
BCH = BINCAP // CHUNK
BCHUNK = 4000
EBCH = E // BCHUNK

_mesh = functools.partial(
    plsc.VectorSubcoreMesh, core_axis_name="c", subcore_axis_name="s",
    num_cores=NC, num_subcores=NS)


@functools.partial(
    pl.kernel,
    out_type=(
        jax.ShapeDtypeStruct((NW, BINCAP), jnp.int32),
        jax.ShapeDtypeStruct((NW, BINCAP), jnp.int32),
        jax.ShapeDtypeStruct((NW, VPT), jnp.float32),
    ),
    mesh=_mesh(),
    compiler_params=pltpu.CompilerParams(needs_layout_passes=False),
    scratch_types=[
        pltpu.VMEM((BCHUNK,), jnp.int32),
        pltpu.VMEM((BCHUNK,), jnp.int32),
        pltpu.VMEM((BCHUNK,), jnp.int32),
        pltpu.VMEM((BCHUNK,), jnp.int32),
        pltpu.VMEM((BINCAP,), jnp.int32),
        pltpu.VMEM((BINCAP,), jnp.int32),
        pltpu.VMEM((VPT,), jnp.float32),
        pltpu.SemaphoreType.DMA,
        pltpu.SemaphoreType.DMA,
    ],
)
def _bin_kernel(src_hbm, dst_hbm, bsrc_out, bdst_out, deg_out,
                sbuf0, sbuf1, dbuf0, dbuf1, bsrc_l, bdst_l, deg_l,
                sem0, sem1):
    c = lax.axis_index("c")
    s = lax.axis_index("s")
    w = c * NS + s
    base = w * VPT
    sems = (sem0, sem1)
    sbufs = (sbuf0, sbuf1)
    dbufs = (dbuf0, dbuf1)

    zi = jnp.zeros((16,), jnp.int32)
    ti = jnp.full((16,), VPT, jnp.int32)
    zf = jnp.zeros((16,), jnp.float32)
    onef = jnp.ones((16,), jnp.float32)

    def _fire(k, p):
        pltpu.async_copy(
            src_hbm.at[pl.ds(k * BCHUNK, BCHUNK)], sbufs[p], sems[p])
        pltpu.async_copy(
            dst_hbm.at[pl.ds(k * BCHUNK, BCHUNK)], dbufs[p], sems[p])

    def _drain(p):
        pltpu.make_async_copy(
            src_hbm.at[pl.ds(0, BCHUNK)], sbufs[p], sems[p]).wait()
        pltpu.make_async_copy(
            dst_hbm.at[pl.ds(0, BCHUNK)], dbufs[p], sems[p]).wait()

    _fire(0, 0)
    _fire(1, 1)

    def pre(i, carry):
        bsrc_l[pl.ds(i * 16, 16)] = zi
        bdst_l[pl.ds(i * 16, 16)] = ti
        return carry

    lax.fori_loop(0, BINCAP // 16, pre, 0)

    def zdeg(i, carry):
        deg_l[pl.ds(i * 16, 16)] = zf
        return carry

    lax.fori_loop(0, VPT // 16, zdeg, 0)

    def _consume(p, cnt):
        for j in range(BCHUNK // 16):
            s16 = sbufs[p][pl.ds(j * 16, 16)]
            d16 = dbufs[p][pl.ds(j * 16, 16)]
            dl16 = d16 - base
            m = (dl16 >= 0) & (dl16 < VPT)
            dlc = jnp.where(m, dl16, 0)
            plsc.addupdate_scatter(deg_l, [dlc], onef, mask=m)
            plsc.store_compressed(bsrc_l.at[pl.ds(cnt, 16)], s16, mask=m)
            plsc.store_compressed(
                bdst_l.at[pl.ds(cnt, 16)], jnp.where(m, dl16, ti), mask=m)
            cnt = jnp.minimum(cnt + jnp.sum(m.astype(jnp.int32)), BINCAP - 16)
        return cnt

    def block(i, cnt):
        for p in range(2):
            k = i * 2 + p
            _drain(p)
            cnt = _consume(p, cnt)

            @pl.when(k + 2 < EBCH)
            def _():
                _fire(k + 2, p)
        return cnt

    lax.fori_loop(0, EBCH // 2, block, jnp.int32(0))

    pltpu.sync_copy(bsrc_l, bsrc_out.at[w])
    pltpu.sync_copy(bdst_l, bdst_out.at[w])
    pltpu.sync_copy(deg_l, deg_out.at[w])


@functools.partial(
    pl.kernel,
    out_type=jax.ShapeDtypeStruct((NPAD, D), jnp.float32),
    mesh=_mesh(),
    compiler_params=pltpu.CompilerParams(needs_layout_passes=False),
    scratch_types=[
        pltpu.VMEM((BINCAP,), jnp.int32),
        pltpu.VMEM((BINCAP,), jnp.int32),
        pltpu.VMEM((CHUNK, D), jnp.float32),
        pltpu.VMEM((CHUNK, D), jnp.float32),
        pltpu.VMEM((ACCR, D), jnp.float32),
        pltpu.SemaphoreType.DMA,
        pltpu.SemaphoreType.DMA,
        pltpu.SemaphoreType.DMA,
        pltpu.SemaphoreType.DMA,
        pltpu.SemaphoreType.DMA,
        pltpu.SemaphoreType.DMA,
        pltpu.SemaphoreType.DMA,
        pltpu.SemaphoreType.DMA,
        pltpu.SemaphoreType.DMA,
    ],
)
def _spmm_kernel(bsrc_hbm, bdst_hbm, g_hbm, out_hbm,
                 sball, dball, rows0, rows1, acc, semi,
                 s00, s01, s02, s03, s10, s11, s12, s13):
    c = lax.axis_index("c")
    s = lax.axis_index("s")
    w = c * NS + s
    sems = ((s00, s01, s02, s03), (s10, s11, s12, s13))
    rowss = (rows0, rows1)

    pltpu.async_copy(bsrc_hbm.at[w], sball, semi)
    pltpu.async_copy(bdst_hbm.at[w], dball, semi)

    zf = jnp.zeros((16,), jnp.float32)

    def zacc(i, carry):
        r = i // (D // 16)
        col = (i % (D // 16)) * 16
        acc[r, pl.ds(col, 16)] = zf
        return carry

    lax.fori_loop(0, ACCR * (D // 16), zacc, 0)

    pltpu.make_async_copy(bsrc_hbm.at[w], sball, semi).wait()
    pltpu.make_async_copy(bdst_hbm.at[w], dball, semi).wait()

    QR = CHUNK // 4

    def _fire(k, p):
        for q in range(4):
            pltpu.async_copy(
                g_hbm.at[sball.at[pl.ds(k * CHUNK + q * QR, QR)]],
                rowss[p].at[pl.ds(q * QR, QR)], sems[p][q])

    def _drain(p):
        for q in range(4):
            pltpu.make_async_copy(
                g_hbm.at[sball.at[pl.ds(0, QR)]],
                rowss[p].at[pl.ds(q * QR, QR)], sems[p][q]).wait()

    _fire(0, 0)
    _fire(1, 1)

    def _consume(k, p):
        for j in range(CHUNK // 16):
            dl16 = dball[pl.ds(k * CHUNK + j * 16, 16)]
            for t in range(16):
                e = j * 16 + t
                dl = dl16[t]
                for cc in range(D // 16):
                    v = rowss[p][e, pl.ds(cc * 16, 16)]
                    plsc.addupdate(acc.at[dl, pl.ds(cc * 16, 16)], v)

    def chunk2(i, carry):
        for p in range(2):
            k = i * 2 + p
            _drain(p)
            _consume(k, p)

            @pl.when(k + 2 < BCH)
            def _():
                _fire(k + 2, p)
        return carry

    lax.fori_loop(0, BCH // 2, chunk2, 0)

    pltpu.sync_copy(acc.at[pl.ds(0, VPT)], out_hbm.at[pl.ds(w * VPT, VPT)])


_RBLK = 1000
_GRID = N // _RBLK

_row_spec = pl.BlockSpec((_RBLK, D), lambda i: (i, 0))
_col_spec = pl.BlockSpec((_RBLK, 1), lambda i: (i, 0))
_w_spec = pl.BlockSpec((D, D), lambda i: (0, 0))
_b_spec = pl.BlockSpec((1, D), lambda i: (0, 0))


def _k1_body(deg_ref, x_ref, w_ref, dinv_ref, g_ref):
    deg = deg_ref[...] + 1.0
    dinv = lax.rsqrt(jnp.maximum(deg, 1.0))
    dinv_ref[...] = dinv
    g_ref[...] = dinv * jnp.dot(x_ref[...], w_ref[...],
                                preferred_element_type=jnp.float32)


_k1 = pl.pallas_call(
    _k1_body,
    grid=(_GRID,),
    in_specs=[_col_spec, _row_spec, _w_spec],
    out_specs=[_col_spec, _row_spec],
    out_shape=[jax.ShapeDtypeStruct((N, 1), jnp.float32),
               jax.ShapeDtypeStruct((N, D), jnp.float32)],
)


def _kmid_body(p_ref, g_ref, dinv_ref, b_ref, w_ref, gn_ref):
    dinv = dinv_ref[...]
    h = dinv * (p_ref[...] + g_ref[...]) + b_ref[...]
    h = jnp.maximum(h, 0.0)
    gn_ref[...] = dinv * jnp.dot(h, w_ref[...],
                                 preferred_element_type=jnp.float32)


_kmid = pl.pallas_call(
    _kmid_body,
    grid=(_GRID,),
    in_specs=[_row_spec, _row_spec, _col_spec, _b_spec, _w_spec],
    out_specs=_row_spec,
    out_shape=jax.ShapeDtypeStruct((N, D), jnp.float32),
)


def _kfin_body(p_ref, g_ref, dinv_ref, b_ref, wo_ref, bo_ref,
               h_ref, out_ref):
    h = dinv_ref[...] * (p_ref[...] + g_ref[...]) + b_ref[...]
    h_ref[...] = h
    out_ref[...] = jnp.dot(h, wo_ref[...],
                           preferred_element_type=jnp.float32) + bo_ref[...]


_kfin = pl.pallas_call(
    _kfin_body,
    grid=(_GRID,),
    in_specs=[_row_spec, _row_spec, _col_spec, _b_spec, _w_spec, _b_spec],
    out_specs=[_row_spec, _row_spec],
    out_shape=[jax.ShapeDtypeStruct((N, D), jnp.float32),
               jax.ShapeDtypeStruct((N, D), jnp.float32)],
)


def kernel(x, edge_index, W1, b1, Wh1, bh1, W2, b2, Wout, bout):
    src = edge_index[0]
    dst = edge_index[1]

    bsrc, bdst, degp = _bin_kernel(src, dst)
    deg = degp.reshape(NPAD, 1)

    dinv, g1 = _k1(deg, x, W1)
    p1 = _spmm_kernel(bsrc, bdst, g1)
    g2 = _kmid(p1, g1, dinv, b1.reshape(1, D), Wh1)
    p2 = _spmm_kernel(bsrc, bdst, g2)
    g3 = _kmid(p2, g2, dinv, bh1.reshape(1, D), W2)
    p3 = _spmm_kernel(bsrc, bdst, g3)

    wout_pad = jnp.zeros((D, D), jnp.float32).at[:, :Wout.shape[1]].set(Wout)
    bout_pad = jnp.zeros((1, D), jnp.float32).at[0, :bout.shape[0]].set(bout)
    h3, out_pad = _kfin(p3, g3, dinv, b2.reshape(1, D), wout_pad, bout_pad)
    return (out_pad[:, :Wout.shape[1]], h3)
